# Initial kernel scaffold; baseline (speedup 1.0000x reference)
import jax
import jax.numpy as jnp
from jax.experimental import pallas as pl


def kernel(x, f_ij, idx_i, idx_j, rcut_ij, W_in2f, W_f1, b_f1, W_f2, b_f2, W_o1, b_o1, W_o2, b_o2):
    raise NotImplementedError("write your pallas kernel here")


# speedup vs baseline: 1.6869x; 1.6869x over previous
import functools

import jax
import jax.numpy as jnp
from jax import lax
from jax.experimental import pallas as pl
from jax.experimental.pallas import tpu as pltpu
from jax.experimental.pallas import tpu_sc as plsc

N, E, D, R = 10000, 320000, 128, 16

_LOG2 = 0.6931471805599453


def _ssp(x):
    return jnp.maximum(x, 0.0) + jnp.log(1.0 + jnp.exp(-jnp.abs(x))) - _LOG2


def _h_body(x_ref, w_ref, o_ref):
    o_ref[...] = jnp.dot(x_ref[...], w_ref[...], preferred_element_type=jnp.float32)


def _h_call(x, w):
    return pl.pallas_call(
        _h_body,
        out_shape=jax.ShapeDtypeStruct((N, D), jnp.float32),
    )(x, w)


_EB = 4000


def _filter_body(f_ref, rc_ref, wf1_ref, bf1_ref, wf2_ref, bf2_ref, o_ref):
    t = jnp.dot(f_ref[...], wf1_ref[...], preferred_element_type=jnp.float32)
    t = _ssp(t + bf1_ref[...])
    w = jnp.dot(t, wf2_ref[...], preferred_element_type=jnp.float32) + bf2_ref[...]
    o_ref[...] = w * rc_ref[...]


def _filter_call(f_ij, rcut_col, wf1, bf1, wf2, bf2):
    grid = E // _EB
    return pl.pallas_call(
        _filter_body,
        grid=(grid,),
        in_specs=[
            pl.BlockSpec((_EB, R), lambda i: (i, 0)),
            pl.BlockSpec((_EB, 1), lambda i: (i, 0)),
            pl.BlockSpec((R, D), lambda i: (0, 0)),
            pl.BlockSpec((1, D), lambda i: (0, 0)),
            pl.BlockSpec((D, D), lambda i: (0, 0)),
            pl.BlockSpec((1, D), lambda i: (0, 0)),
        ],
        out_specs=pl.BlockSpec((_EB, D), lambda i: (i, 0)),
        out_shape=jax.ShapeDtypeStruct((E, D), jnp.float32),
    )(f_ij, rcut_col, wf1, bf1, wf2, bf2)


_NC, _NS = 2, 16
_NW = _NC * _NS
_EPW = E // _NW
_K = 80
_NCHUNK = _EPW // _K
_NP = 10240
_RPT = _NP // _NS
_ZR = 128


def _sc_body(h_hbm, wij_hbm, idxj_hbm, idxi_hbm, out_hbm,
             idxj_v, idxi_v, rows_v, wij_v, bounce_v, agg_sh, sem):
    c = lax.axis_index("c")
    s = lax.axis_index("s")
    wid = s * _NC + c

    def _z1(i, carry):
        r = i // (D // 16)
        col = (i % (D // 16)) * 16
        bounce_v[r, pl.ds(col, 16)] = jnp.zeros((16,), jnp.float32)
        return carry

    lax.fori_loop(0, _ZR * (D // 16), _z1, 0)
    r0 = s * _RPT

    def _zcopy(j, carry):
        pltpu.sync_copy(bounce_v, agg_sh.at[pl.ds(r0 + j * _ZR, _ZR)])
        return carry

    lax.fori_loop(0, _RPT // _ZR, _zcopy, 0)
    plsc.subcore_barrier()

    ebase0 = wid * _EPW

    def _chunk(k, carry):
        eb = ebase0 + k * _K
        pltpu.sync_copy(idxj_hbm.at[pl.ds(eb, _K)], idxj_v)
        pltpu.sync_copy(idxi_hbm.at[pl.ds(eb, _K)], idxi_v)
        gat = pltpu.async_copy(h_hbm.at[idxj_v], rows_v, sem)
        pltpu.sync_copy(wij_hbm.at[pl.ds(eb, _K)], wij_v)
        gat.wait()

        def _mul(i, cc):
            e = i // (D // 16)
            col = (i % (D // 16)) * 16
            rows_v[e, pl.ds(col, 16)] = (
                rows_v[e, pl.ds(col, 16)] * wij_v[e, pl.ds(col, 16)])
            return cc

        lax.fori_loop(0, _K * (D // 16), _mul, 0)
        pltpu.sync_copy(rows_v, agg_sh.at[idxi_v], add=True)
        return carry

    lax.fori_loop(0, _NCHUNK, _chunk, 0)
    plsc.subcore_barrier()

    def _dump(j, carry):
        rr = r0 + j * _ZR
        pltpu.sync_copy(agg_sh.at[pl.ds(rr, _ZR)], bounce_v)
        pltpu.sync_copy(bounce_v, out_hbm.at[pl.ds(c * _NP + rr, _ZR)])
        return carry

    lax.fori_loop(0, _RPT // _ZR, _dump, 0)


def _sc_conv(h, wij, idx_j, idx_i):
    mesh = plsc.VectorSubcoreMesh(core_axis_name="c", subcore_axis_name="s")
    fn = pl.kernel(
        _sc_body,
        out_type=jax.ShapeDtypeStruct((_NC * _NP, D), jnp.float32),
        mesh=mesh,
        scratch_types=[
            pltpu.VMEM((_K,), jnp.int32),
            pltpu.VMEM((_K,), jnp.int32),
            pltpu.VMEM((_K, D), jnp.float32),
            pltpu.VMEM((_K, D), jnp.float32),
            pltpu.VMEM((_ZR, D), jnp.float32),
            pltpu.VMEM_SHARED((_NP, D), jnp.float32),
            pltpu.SemaphoreType.DMA,
        ],
    )
    return fn(h, wij, idx_j, idx_i)


def _out_body(p_ref, wo1_ref, bo1_ref, wo2_ref, bo2_ref, o_ref):
    agg = p_ref[:N, :] + p_ref[_NP:_NP + N, :]
    t = _ssp(jnp.dot(agg, wo1_ref[...], preferred_element_type=jnp.float32)
             + bo1_ref[...])
    o_ref[...] = (jnp.dot(t, wo2_ref[...], preferred_element_type=jnp.float32)
                  + bo2_ref[...])


def _out_call(partials, wo1, bo1, wo2, bo2):
    return pl.pallas_call(
        _out_body,
        out_shape=jax.ShapeDtypeStruct((N, D), jnp.float32),
    )(partials, wo1, bo1, wo2, bo2)


def kernel(x, f_ij, idx_i, idx_j, rcut_ij,
           W_in2f, W_f1, b_f1, W_f2, b_f2, W_o1, b_o1, W_o2, b_o2):
    idx_i32 = idx_i.astype(jnp.int32)
    idx_j32 = idx_j.astype(jnp.int32)
    h = _h_call(x, W_in2f)
    wij = _filter_call(f_ij, rcut_ij.reshape(E, 1),
                       W_f1, b_f1.reshape(1, D), W_f2, b_f2.reshape(1, D))
    partials = _sc_conv(h, wij, idx_j32, idx_i32)
    return _out_call(partials, W_o1, b_o1.reshape(1, D),
                     W_o2, b_o2.reshape(1, D))


# speedup vs baseline: 1.8097x; 1.0728x over previous
import functools

import jax
import jax.numpy as jnp
from jax import lax
from jax.experimental import pallas as pl
from jax.experimental.pallas import tpu as pltpu
from jax.experimental.pallas import tpu_sc as plsc

N, E, D, R = 10000, 320000, 128, 16

_LOG2 = 0.6931471805599453


def _ssp(x):
    return jnp.maximum(x, 0.0) + jnp.log(1.0 + jnp.exp(-jnp.abs(x))) - _LOG2


def _h_body(x_ref, w_ref, o_ref):
    o_ref[...] = jnp.dot(x_ref[...], w_ref[...], preferred_element_type=jnp.float32)


def _h_call(x, w):
    return pl.pallas_call(
        _h_body,
        out_shape=jax.ShapeDtypeStruct((N, D), jnp.float32),
    )(x, w)


_EB = 4000


def _filter_body(f_ref, rc_ref, wf1_ref, bf1_ref, wf2_ref, bf2_ref, o_ref):
    t = jnp.dot(f_ref[...], wf1_ref[...], preferred_element_type=jnp.float32)
    t = _ssp(t + bf1_ref[...])
    w = jnp.dot(t, wf2_ref[...], preferred_element_type=jnp.float32) + bf2_ref[...]
    o_ref[...] = w * rc_ref[...]


def _filter_call(f_ij, rcut_col, wf1, bf1, wf2, bf2):
    grid = E // _EB
    return pl.pallas_call(
        _filter_body,
        grid=(grid,),
        in_specs=[
            pl.BlockSpec((_EB, R), lambda i: (i, 0)),
            pl.BlockSpec((_EB, 1), lambda i: (i, 0)),
            pl.BlockSpec((R, D), lambda i: (0, 0)),
            pl.BlockSpec((1, D), lambda i: (0, 0)),
            pl.BlockSpec((D, D), lambda i: (0, 0)),
            pl.BlockSpec((1, D), lambda i: (0, 0)),
        ],
        out_specs=pl.BlockSpec((_EB, D), lambda i: (i, 0)),
        out_shape=jax.ShapeDtypeStruct((E, D), jnp.float32),
    )(f_ij, rcut_col, wf1, bf1, wf2, bf2)


_NC, _NS = 2, 16
_NW = _NC * _NS
_EPW = E // _NW
_K = 80
_NCHUNK = _EPW // _K
_NP = 10240
_RPT = _NP // _NS
_ZR = 128


_NB = 2


def _sc_body(h_hbm, wij_hbm, idxj_hbm, idxi_hbm, out_hbm,
             idxj_c, idxi_c, rows_v, wij_v, agg_sh,
             isem, g0, g1):
    gsems = (g0, g1)
    c = lax.axis_index("c")
    s = lax.axis_index("s")
    wid = s * _NC + c

    def _z1(i, carry):
        r = i // (D // 16)
        col = (i % (D // 16)) * 16
        wij_v[r, pl.ds(col, 16)] = jnp.zeros((16,), jnp.float32)
        return carry

    lax.fori_loop(0, _ZR * (D // 16), _z1, 0)
    r0 = s * _RPT
    zsrc = wij_v.at[pl.ds(0, _ZR)]

    def _zcopy(j, carry):
        pltpu.sync_copy(zsrc, agg_sh.at[pl.ds(r0 + j * _ZR, _ZR)])
        return carry

    lax.fori_loop(0, _RPT // _ZR, _zcopy, 0)
    plsc.subcore_barrier()

    ebase0 = wid * _EPW
    cb0 = wid * _NCHUNK

    def _mul_chunk(b):
        def _mul(i, cc, _b=b):
            e = _b * _K + i // (D // 16)
            col = (i % (D // 16)) * 16
            rows_v[e, pl.ds(col, 16)] = (
                rows_v[e, pl.ds(col, 16)] * wij_v[e, pl.ds(col, 16)])
            return cc

        lax.fori_loop(0, _K * (D // 16), _mul, 0, unroll=8)

    def _block(blk, carry):
        k0 = blk * _NB
        for b in range(_NB):
            pltpu.async_copy(idxj_hbm.at[cb0 + k0 + b], idxj_c.at[b], isem)
            pltpu.async_copy(idxi_hbm.at[cb0 + k0 + b], idxi_c.at[b], isem)
        for b in range(_NB):
            pltpu.make_async_copy(idxj_hbm.at[cb0], idxj_c.at[b], isem).wait()
            pltpu.make_async_copy(idxj_hbm.at[cb0], idxi_c.at[b], isem).wait()
        for b in range(_NB):
            pltpu.async_copy(h_hbm.at[idxj_c.at[b, 0]],
                             rows_v.at[pl.ds(b * _K, _K)], gsems[b])
        wij_d = pltpu.async_copy(
            wij_hbm.at[pl.ds(ebase0 + k0 * _K, _NB * _K)], wij_v, isem)

        pltpu.make_async_copy(h_hbm.at[pl.ds(0, _K)],
                              rows_v.at[pl.ds(0, _K)], gsems[0]).wait()
        wij_d.wait()
        _mul_chunk(0)
        pltpu.sync_copy(rows_v.at[pl.ds(0, _K)],
                        agg_sh.at[idxi_c.at[0, 0]], add=True)

        pltpu.make_async_copy(h_hbm.at[pl.ds(0, _K)],
                              rows_v.at[pl.ds(_K, _K)], gsems[1]).wait()
        _mul_chunk(1)
        pltpu.sync_copy(rows_v.at[pl.ds(_K, _K)],
                        agg_sh.at[idxi_c.at[1, 0]], add=True)
        return carry

    lax.fori_loop(0, _NCHUNK // _NB, _block, 0)

    kt = _NCHUNK - 1
    pltpu.sync_copy(idxj_hbm.at[cb0 + kt], idxj_c.at[0])
    pltpu.sync_copy(idxi_hbm.at[cb0 + kt], idxi_c.at[0])
    pltpu.async_copy(h_hbm.at[idxj_c.at[0, 0]],
                     rows_v.at[pl.ds(0, _K)], gsems[0])
    pltpu.sync_copy(wij_hbm.at[pl.ds(ebase0 + kt * _K, _K)],
                    wij_v.at[pl.ds(0, _K)])
    pltpu.make_async_copy(h_hbm.at[pl.ds(0, _K)],
                          rows_v.at[pl.ds(0, _K)], gsems[0]).wait()
    _mul_chunk(0)
    pltpu.sync_copy(rows_v.at[pl.ds(0, _K)],
                    agg_sh.at[idxi_c.at[0, 0]], add=True)
    plsc.subcore_barrier()

    def _dump(j, carry):
        rr = r0 + j * _ZR
        pltpu.sync_copy(agg_sh.at[pl.ds(rr, _ZR)], zsrc)
        pltpu.sync_copy(zsrc, out_hbm.at[pl.ds(c * _NP + rr, _ZR)])
        return carry

    lax.fori_loop(0, _RPT // _ZR, _dump, 0)


def _sc_conv(h, wij, idx_j, idx_i):
    mesh = plsc.VectorSubcoreMesh(core_axis_name="c", subcore_axis_name="s")
    fn = pl.kernel(
        _sc_body,
        out_type=jax.ShapeDtypeStruct((_NC * _NP, D), jnp.float32),
        mesh=mesh,
        scratch_types=[
            pltpu.VMEM((_NB, 1, _K), jnp.int32),
            pltpu.VMEM((_NB, 1, _K), jnp.int32),
            pltpu.VMEM((_NB * _K, D), jnp.float32),
            pltpu.VMEM((_NB * _K, D), jnp.float32),
            pltpu.VMEM_SHARED((_NP, D), jnp.float32),
            pltpu.SemaphoreType.DMA,
            pltpu.SemaphoreType.DMA,
            pltpu.SemaphoreType.DMA,
        ],
    )
    return fn(h, wij, idx_j.reshape(E // _K, 1, _K),
              idx_i.reshape(E // _K, 1, _K))


def _out_body(p_ref, wo1_ref, bo1_ref, wo2_ref, bo2_ref, o_ref):
    agg = p_ref[:N, :] + p_ref[_NP:_NP + N, :]
    t = _ssp(jnp.dot(agg, wo1_ref[...], preferred_element_type=jnp.float32)
             + bo1_ref[...])
    o_ref[...] = (jnp.dot(t, wo2_ref[...], preferred_element_type=jnp.float32)
                  + bo2_ref[...])


def _out_call(partials, wo1, bo1, wo2, bo2):
    return pl.pallas_call(
        _out_body,
        out_shape=jax.ShapeDtypeStruct((N, D), jnp.float32),
    )(partials, wo1, bo1, wo2, bo2)


def kernel(x, f_ij, idx_i, idx_j, rcut_ij,
           W_in2f, W_f1, b_f1, W_f2, b_f2, W_o1, b_o1, W_o2, b_o2):
    idx_i32 = idx_i.astype(jnp.int32)
    idx_j32 = idx_j.astype(jnp.int32)
    h = _h_call(x, W_in2f)
    wij = _filter_call(f_ij, rcut_ij.reshape(E, 1),
                       W_f1, b_f1.reshape(1, D), W_f2, b_f2.reshape(1, D))
    partials = _sc_conv(h, wij, idx_j32, idx_i32)
    return _out_call(partials, W_o1, b_o1.reshape(1, D),
                     W_o2, b_o2.reshape(1, D))


# speedup vs baseline: 2.0237x; 1.1183x over previous
import functools

import jax
import jax.numpy as jnp
from jax import lax
from jax.experimental import pallas as pl
from jax.experimental.pallas import tpu as pltpu
from jax.experimental.pallas import tpu_sc as plsc

N, E, D, R = 10000, 320000, 128, 16

_LOG2 = 0.6931471805599453


def _ssp(x):
    return jnp.maximum(x, 0.0) + jnp.log(1.0 + jnp.exp(-jnp.abs(x))) - _LOG2


def _h_body(x_ref, w_ref, o_ref):
    o_ref[...] = jnp.dot(x_ref[...], w_ref[...], preferred_element_type=jnp.float32)


def _h_call(x, w):
    return pl.pallas_call(
        _h_body,
        out_shape=jax.ShapeDtypeStruct((N, D), jnp.float32),
    )(x, w)


_EB = 4000


def _filter_body(f_ref, rc_ref, wf1_ref, bf1_ref, wf2_ref, bf2_ref, o_ref):
    t = jnp.dot(f_ref[...], wf1_ref[...], preferred_element_type=jnp.float32)
    t = _ssp(t + bf1_ref[...])
    w = jnp.dot(t, wf2_ref[...], preferred_element_type=jnp.float32) + bf2_ref[...]
    o_ref[...] = w * rc_ref[...]


def _filter_call(f_ij, rcut_col, wf1, bf1, wf2, bf2):
    grid = E // _EB
    return pl.pallas_call(
        _filter_body,
        grid=(grid,),
        in_specs=[
            pl.BlockSpec((_EB, R), lambda i: (i, 0)),
            pl.BlockSpec((_EB, 1), lambda i: (i, 0)),
            pl.BlockSpec((R, D), lambda i: (0, 0)),
            pl.BlockSpec((1, D), lambda i: (0, 0)),
            pl.BlockSpec((D, D), lambda i: (0, 0)),
            pl.BlockSpec((1, D), lambda i: (0, 0)),
        ],
        out_specs=pl.BlockSpec((_EB, D), lambda i: (i, 0)),
        out_shape=jax.ShapeDtypeStruct((E, D), jnp.float32),
    )(f_ij, rcut_col, wf1, bf1, wf2, bf2)


_NC, _NS = 2, 16
_NW = _NC * _NS
_EPW = E // _NW
_K = 80
_NCHUNK = _EPW // _K
_NP = 10240
_RPT = _NP // _NS
_ZR = 128
_NPAIR = (_NCHUNK - 1) // 2


def _sc_body(h_hbm, wij_hbm, comb_hbm, out_hbm,
             comb_v, idxj0, idxj1, idxi0, idxi1, rows_v, wij_v, agg_sh,
             i0sem, i1sem, g0, g1, w0, w1, s0, s1):
    c = lax.axis_index("c")
    s = lax.axis_index("s")
    wid = s * _NC + c

    def _z1(i, carry):
        r = i // (D // 16)
        col = (i % (D // 16)) * 16
        wij_v[r, pl.ds(col, 16)] = jnp.zeros((16,), jnp.float32)
        return carry

    lax.fori_loop(0, _ZR * (D // 16), _z1, 0)
    r0 = s * _RPT
    zsrc = wij_v.at[pl.ds(0, _ZR)]

    def _zcopy(j, carry):
        pltpu.sync_copy(zsrc, agg_sh.at[pl.ds(r0 + j * _ZR, _ZR)])
        return carry

    lax.fori_loop(0, _RPT // _ZR, _zcopy, 0)
    plsc.subcore_barrier()

    ebase0 = wid * _EPW
    cb0 = wid * _NCHUNK

    def _extract(slot, dst, base):
        def _cp(i, cc):
            dst[pl.ds(i * 16, 16)] = comb_v[slot, 0, pl.ds(base + i * 16, 16)]
            return cc

        lax.fori_loop(0, _K // 16, _cp, 0, unroll=True)

    def _fire(k, slot, idxj, gsem, wsem):
        pltpu.async_copy(h_hbm.at[idxj], rows_v.at[pl.ds(slot * _K, _K)], gsem)
        pltpu.async_copy(wij_hbm.at[pl.ds(ebase0 + k * _K, _K)],
                         wij_v.at[pl.ds(slot * _K, _K)], wsem)

    def _wait_gw(slot, gsem, wsem):
        pltpu.make_async_copy(h_hbm.at[pl.ds(0, _K)],
                              rows_v.at[pl.ds(slot * _K, _K)], gsem).wait()
        pltpu.make_async_copy(wij_hbm.at[pl.ds(0, _K)],
                              wij_v.at[pl.ds(slot * _K, _K)], wsem).wait()

    def _drain_s(sem):
        pltpu.make_async_copy(h_hbm.at[pl.ds(0, _K)],
                              rows_v.at[pl.ds(0, _K)], sem).wait()

    def _mul_chunk(slot):
        def _mul(i, cc):
            e = slot * _K + i // (D // 16)
            col = (i % (D // 16)) * 16
            rows_v[e, pl.ds(col, 16)] = (
                rows_v[e, pl.ds(col, 16)] * wij_v[e, pl.ds(col, 16)])
            return cc

        lax.fori_loop(0, _K * (D // 16), _mul, 0, unroll=8)

    pltpu.sync_copy(comb_hbm.at[cb0], comb_v.at[0])
    pltpu.sync_copy(comb_hbm.at[cb0 + 1], comb_v.at[1])
    _extract(0, idxj0, 0)
    _extract(0, idxi0, _K)
    _extract(1, idxj1, 0)
    _extract(1, idxi1, _K)
    _fire(0, 0, idxj0, g0, w0)
    _fire(1, 1, idxj1, g1, w1)

    def _pair(blk, carry):
        k = 2 * blk
        pltpu.async_copy(comb_hbm.at[cb0 + k + 2], comb_v.at[0], i0sem)

        @pl.when(blk < _NPAIR - 1)
        def _():
            pltpu.async_copy(comb_hbm.at[cb0 + k + 3], comb_v.at[1], i1sem)

        _wait_gw(0, g0, w0)
        _mul_chunk(0)
        pltpu.async_copy(rows_v.at[pl.ds(0, _K)], agg_sh.at[idxi0], s0,
                         add=True)

        _wait_gw(1, g1, w1)
        _mul_chunk(1)
        pltpu.async_copy(rows_v.at[pl.ds(_K, _K)], agg_sh.at[idxi1], s1,
                         add=True)

        pltpu.make_async_copy(comb_hbm.at[cb0], comb_v.at[0], i0sem).wait()
        _drain_s(s0)
        _extract(0, idxj0, 0)
        _extract(0, idxi0, _K)
        _fire(k + 2, 0, idxj0, g0, w0)

        @pl.when(blk < _NPAIR - 1)
        def _():
            pltpu.make_async_copy(comb_hbm.at[cb0], comb_v.at[1], i1sem).wait()
            _drain_s(s1)
            _extract(1, idxj1, 0)
            _extract(1, idxi1, _K)
            _fire(k + 3, 1, idxj1, g1, w1)

        return carry

    lax.fori_loop(0, _NPAIR, _pair, 0)

    _wait_gw(0, g0, w0)
    _mul_chunk(0)
    pltpu.sync_copy(rows_v.at[pl.ds(0, _K)], agg_sh.at[idxi0], add=True)
    _drain_s(s1)
    plsc.subcore_barrier()

    def _dump(j, carry):
        rr = r0 + j * _ZR
        pltpu.sync_copy(agg_sh.at[pl.ds(rr, _ZR)], zsrc)
        pltpu.sync_copy(zsrc, out_hbm.at[pl.ds(c * _NP + rr, _ZR)])
        return carry

    lax.fori_loop(0, _RPT // _ZR, _dump, 0)


def _sc_conv(h, wij, comb):
    mesh = plsc.VectorSubcoreMesh(core_axis_name="c", subcore_axis_name="s")
    fn = pl.kernel(
        _sc_body,
        out_type=jax.ShapeDtypeStruct((_NC * _NP, D), jnp.float32),
        mesh=mesh,
        scratch_types=[
            pltpu.VMEM((2, 1, 2 * _K), jnp.int32),
            pltpu.VMEM((_K,), jnp.int32),
            pltpu.VMEM((_K,), jnp.int32),
            pltpu.VMEM((_K,), jnp.int32),
            pltpu.VMEM((_K,), jnp.int32),
            pltpu.VMEM((2 * _K, D), jnp.float32),
            pltpu.VMEM((2 * _K, D), jnp.float32),
            pltpu.VMEM_SHARED((_NP, D), jnp.float32),
            pltpu.SemaphoreType.DMA,
            pltpu.SemaphoreType.DMA,
            pltpu.SemaphoreType.DMA,
            pltpu.SemaphoreType.DMA,
            pltpu.SemaphoreType.DMA,
            pltpu.SemaphoreType.DMA,
            pltpu.SemaphoreType.DMA,
            pltpu.SemaphoreType.DMA,
        ],
    )
    return fn(h, wij, comb)


def _out_body(p_ref, wo1_ref, bo1_ref, wo2_ref, bo2_ref, o_ref):
    agg = p_ref[:N, :] + p_ref[_NP:_NP + N, :]
    t = _ssp(jnp.dot(agg, wo1_ref[...], preferred_element_type=jnp.float32)
             + bo1_ref[...])
    o_ref[...] = (jnp.dot(t, wo2_ref[...], preferred_element_type=jnp.float32)
                  + bo2_ref[...])


def _out_call(partials, wo1, bo1, wo2, bo2):
    return pl.pallas_call(
        _out_body,
        out_shape=jax.ShapeDtypeStruct((N, D), jnp.float32),
    )(partials, wo1, bo1, wo2, bo2)


def kernel(x, f_ij, idx_i, idx_j, rcut_ij,
           W_in2f, W_f1, b_f1, W_f2, b_f2, W_o1, b_o1, W_o2, b_o2):
    idx_i32 = idx_i.astype(jnp.int32)
    idx_j32 = idx_j.astype(jnp.int32)
    comb = jnp.concatenate(
        [idx_j32.reshape(E // _K, 1, _K), idx_i32.reshape(E // _K, 1, _K)],
        axis=2)
    h = _h_call(x, W_in2f)
    wij = _filter_call(f_ij, rcut_ij.reshape(E, 1),
                       W_f1, b_f1.reshape(1, D), W_f2, b_f2.reshape(1, D))
    partials = _sc_conv(h, wij, comb)
    return _out_call(partials, W_o1, b_o1.reshape(1, D),
                     W_o2, b_o2.reshape(1, D))


# speedup vs baseline: 2.0675x; 1.0216x over previous
import functools

import jax
import jax.numpy as jnp
from jax import lax
from jax.experimental import pallas as pl
from jax.experimental.pallas import tpu as pltpu
from jax.experimental.pallas import tpu_sc as plsc

N, E, D, R = 10000, 320000, 128, 16

_LOG2 = 0.6931471805599453


def _ssp(x):
    return jnp.maximum(x, 0.0) + jnp.log(1.0 + jnp.exp(-jnp.abs(x))) - _LOG2


def _pack_cols(w):
    def rne(x):
        r = jax.lax.bitcast_convert_type(x, jnp.uint32)
        return (r + jnp.uint32(0x7FFF) + ((r >> 16) & jnp.uint32(1))) >> 16

    parts = []
    for g in range(4):
        lo = rne(w[..., 32 * g:32 * g + 16])
        hi = rne(w[..., 32 * g + 16:32 * g + 32])
        parts.append((hi << 16) | lo)
    return jax.lax.bitcast_convert_type(jnp.concatenate(parts, axis=-1),
                                        jnp.int32)


def _h_body(x_ref, w_ref, o_ref):
    o_ref[...] = jnp.dot(x_ref[...], w_ref[...], preferred_element_type=jnp.float32)


def _h_call(x, w):
    return pl.pallas_call(
        _h_body,
        out_shape=jax.ShapeDtypeStruct((N, D), jnp.float32),
    )(x, w)


_EB = 4000


def _filter_body(f_ref, rc_ref, wf1_ref, bf1_ref, wf2_ref, bf2_ref, o_ref):
    t = jnp.dot(f_ref[...], wf1_ref[...], preferred_element_type=jnp.float32)
    t = _ssp(t + bf1_ref[...])
    w = jnp.dot(t, wf2_ref[...], preferred_element_type=jnp.float32) + bf2_ref[...]
    w4 = (w * rc_ref[...]).reshape(_EB // 80, 2, 40, D)
    lo = _pack_cols(w4[:, 0])
    hi = _pack_cols(w4[:, 1])
    o_ref[...] = jnp.concatenate([lo, hi], axis=-1).reshape(_EB // 2, D)


def _filter_call(f_ij, rcut_col, wf1, bf1, wf2, bf2):
    grid = E // _EB
    return pl.pallas_call(
        _filter_body,
        grid=(grid,),
        in_specs=[
            pl.BlockSpec((_EB, R), lambda i: (i, 0)),
            pl.BlockSpec((_EB, 1), lambda i: (i, 0)),
            pl.BlockSpec((R, D), lambda i: (0, 0)),
            pl.BlockSpec((1, D), lambda i: (0, 0)),
            pl.BlockSpec((D, D), lambda i: (0, 0)),
            pl.BlockSpec((1, D), lambda i: (0, 0)),
        ],
        out_specs=pl.BlockSpec((_EB // 2, D), lambda i: (i, 0)),
        out_shape=jax.ShapeDtypeStruct((E // 2, D), jnp.int32),
    )(f_ij, rcut_col, wf1, bf1, wf2, bf2)


_NC, _NS = 2, 16
_NW = _NC * _NS
_EPW = E // _NW
_K = 80
_NCHUNK = _EPW // _K
_NP = 10240
_RPT = _NP // _NS
_ZR = 128
_NPAIR = (_NCHUNK - 1) // 2


def _sc_body(h_hbm, wij_hbm, comb_hbm, out_hbm,
             comb_v, idxj0, idxj1, idxi0, idxi1, rows_v, wij_v, agg_sh,
             i0sem, i1sem, g0, g1, w0, w1, s0, s1):
    c = lax.axis_index("c")
    s = lax.axis_index("s")
    wid = s * _NC + c

    def _z1(i, carry):
        r = i // (D // 16)
        col = (i % (D // 16)) * 16
        rows_v[r, pl.ds(col, 16)] = jnp.zeros((16,), jnp.float32)
        return carry

    lax.fori_loop(0, _ZR * (D // 16), _z1, 0)
    r0 = s * _RPT
    zsrc = rows_v.at[pl.ds(0, _ZR)]

    def _zcopy(j, carry):
        pltpu.sync_copy(zsrc, agg_sh.at[pl.ds(r0 + j * _ZR, _ZR)])
        return carry

    lax.fori_loop(0, _RPT // _ZR, _zcopy, 0)
    plsc.subcore_barrier()

    ebase0 = wid * _EPW
    cb0 = wid * _NCHUNK

    def _extract(slot, dst, base):
        def _cp(i, cc):
            dst[pl.ds(i * 16, 16)] = comb_v[slot, 0, pl.ds(base + i * 16, 16)]
            return cc

        lax.fori_loop(0, _K // 16, _cp, 0, unroll=True)

    _KW = _K // 2

    def _fire(k, slot, idxj, gsem, wsem):
        pltpu.async_copy(h_hbm.at[idxj], rows_v.at[pl.ds(slot * _K, _K)], gsem)
        woff = pl.multiple_of((ebase0 + k * _K) // 2, 8)
        pltpu.async_copy(wij_hbm.at[pl.ds(woff, _KW)],
                         wij_v.at[pl.ds(slot * _KW, _KW)], wsem)

    def _wait_gw(slot, gsem, wsem):
        pltpu.make_async_copy(h_hbm.at[pl.ds(0, _K)],
                              rows_v.at[pl.ds(slot * _K, _K)], gsem).wait()
        pltpu.make_async_copy(wij_hbm.at[pl.ds(0, _KW)],
                              wij_v.at[pl.ds(slot * _KW, _KW)], wsem).wait()

    def _drain_s(sem):
        pltpu.make_async_copy(h_hbm.at[pl.ds(0, _K)],
                              rows_v.at[pl.ds(0, _K)], sem).wait()

    def _mul_chunk(slot):
        def _mul(i, cc):
            e = i // 4
            g = i % 4
            re = slot * _K + e
            ww = wij_v[slot * _KW + e % _KW,
                       pl.ds((e // _KW) * 64 + g * 16, 16)]
            wlo = lax.bitcast_convert_type(ww << 16, jnp.float32)
            whi = lax.bitcast_convert_type(ww & (-65536), jnp.float32)
            rows_v[re, pl.ds(32 * g, 16)] = (
                rows_v[re, pl.ds(32 * g, 16)] * wlo)
            rows_v[re, pl.ds(32 * g + 16, 16)] = (
                rows_v[re, pl.ds(32 * g + 16, 16)] * whi)
            return cc

        lax.fori_loop(0, _K * 4, _mul, 0, unroll=8)

    pltpu.sync_copy(comb_hbm.at[cb0], comb_v.at[0])
    pltpu.sync_copy(comb_hbm.at[cb0 + 1], comb_v.at[1])
    _extract(0, idxj0, 0)
    _extract(0, idxi0, _K)
    _extract(1, idxj1, 0)
    _extract(1, idxi1, _K)
    _fire(0, 0, idxj0, g0, w0)
    _fire(1, 1, idxj1, g1, w1)

    def _pair(blk, carry):
        k = 2 * blk
        pltpu.async_copy(comb_hbm.at[cb0 + k + 2], comb_v.at[0], i0sem)

        @pl.when(blk < _NPAIR - 1)
        def _():
            pltpu.async_copy(comb_hbm.at[cb0 + k + 3], comb_v.at[1], i1sem)

        _wait_gw(0, g0, w0)
        _mul_chunk(0)
        pltpu.async_copy(rows_v.at[pl.ds(0, _K)], agg_sh.at[idxi0], s0,
                         add=True)

        _wait_gw(1, g1, w1)
        _mul_chunk(1)
        pltpu.async_copy(rows_v.at[pl.ds(_K, _K)], agg_sh.at[idxi1], s1,
                         add=True)

        pltpu.make_async_copy(comb_hbm.at[cb0], comb_v.at[0], i0sem).wait()
        _drain_s(s0)
        _extract(0, idxj0, 0)
        _extract(0, idxi0, _K)
        _fire(k + 2, 0, idxj0, g0, w0)

        @pl.when(blk < _NPAIR - 1)
        def _():
            pltpu.make_async_copy(comb_hbm.at[cb0], comb_v.at[1], i1sem).wait()
            _drain_s(s1)
            _extract(1, idxj1, 0)
            _extract(1, idxi1, _K)
            _fire(k + 3, 1, idxj1, g1, w1)

        return carry

    lax.fori_loop(0, _NPAIR, _pair, 0)

    _wait_gw(0, g0, w0)
    _mul_chunk(0)
    pltpu.sync_copy(rows_v.at[pl.ds(0, _K)], agg_sh.at[idxi0], add=True)
    _drain_s(s1)
    plsc.subcore_barrier()

    def _dump(j, carry):
        rr = r0 + j * _ZR
        pltpu.sync_copy(agg_sh.at[pl.ds(rr, _ZR)], zsrc)
        pltpu.sync_copy(zsrc, out_hbm.at[pl.ds(c * _NP + rr, _ZR)])
        return carry

    lax.fori_loop(0, _RPT // _ZR, _dump, 0)


def _sc_conv(h, wij, comb):
    mesh = plsc.VectorSubcoreMesh(core_axis_name="c", subcore_axis_name="s")
    fn = pl.kernel(
        _sc_body,
        out_type=jax.ShapeDtypeStruct((_NC * _NP, D), jnp.float32),
        mesh=mesh,
        scratch_types=[
            pltpu.VMEM((2, 1, 2 * _K), jnp.int32),
            pltpu.VMEM((_K,), jnp.int32),
            pltpu.VMEM((_K,), jnp.int32),
            pltpu.VMEM((_K,), jnp.int32),
            pltpu.VMEM((_K,), jnp.int32),
            pltpu.VMEM((2 * _K, D), jnp.float32),
            pltpu.VMEM((_K, D), jnp.int32),
            pltpu.VMEM_SHARED((_NP, D), jnp.float32),
            pltpu.SemaphoreType.DMA,
            pltpu.SemaphoreType.DMA,
            pltpu.SemaphoreType.DMA,
            pltpu.SemaphoreType.DMA,
            pltpu.SemaphoreType.DMA,
            pltpu.SemaphoreType.DMA,
            pltpu.SemaphoreType.DMA,
            pltpu.SemaphoreType.DMA,
        ],
    )
    return fn(h, wij, comb)


def _out_body(p_ref, wo1_ref, bo1_ref, wo2_ref, bo2_ref, o_ref):
    agg = p_ref[:N, :] + p_ref[_NP:_NP + N, :]
    t = _ssp(jnp.dot(agg, wo1_ref[...], preferred_element_type=jnp.float32)
             + bo1_ref[...])
    o_ref[...] = (jnp.dot(t, wo2_ref[...], preferred_element_type=jnp.float32)
                  + bo2_ref[...])


def _out_call(partials, wo1, bo1, wo2, bo2):
    return pl.pallas_call(
        _out_body,
        out_shape=jax.ShapeDtypeStruct((N, D), jnp.float32),
    )(partials, wo1, bo1, wo2, bo2)


def kernel(x, f_ij, idx_i, idx_j, rcut_ij,
           W_in2f, W_f1, b_f1, W_f2, b_f2, W_o1, b_o1, W_o2, b_o2):
    idx_i32 = idx_i.astype(jnp.int32)
    idx_j32 = idx_j.astype(jnp.int32)
    comb = jnp.concatenate(
        [idx_j32.reshape(E // _K, 1, _K), idx_i32.reshape(E // _K, 1, _K)],
        axis=2)
    h = _h_call(x, W_in2f)
    wij = _filter_call(f_ij, rcut_ij.reshape(E, 1),
                       W_f1, b_f1.reshape(1, D), W_f2, b_f2.reshape(1, D))
    partials = _sc_conv(h, wij, comb)
    return _out_call(partials, W_o1, b_o1.reshape(1, D),
                     W_o2, b_o2.reshape(1, D))


# speedup vs baseline: 2.0708x; 1.0016x over previous
import functools

import jax
import jax.numpy as jnp
from jax import lax
from jax.experimental import pallas as pl
from jax.experimental.pallas import tpu as pltpu
from jax.experimental.pallas import tpu_sc as plsc

N, E, D, R = 10000, 320000, 128, 16

_LOG2 = 0.6931471805599453


def _ssp(x):
    return jnp.maximum(x, 0.0) + jnp.log(1.0 + jnp.exp(-jnp.abs(x))) - _LOG2


def _pack_cols(w):
    def rne(x):
        r = jax.lax.bitcast_convert_type(x, jnp.uint32)
        return (r + jnp.uint32(0x7FFF) + ((r >> 16) & jnp.uint32(1))) >> 16

    parts = []
    for g in range(4):
        lo = rne(w[..., 32 * g:32 * g + 16])
        hi = rne(w[..., 32 * g + 16:32 * g + 32])
        parts.append((hi << 16) | lo)
    return jax.lax.bitcast_convert_type(jnp.concatenate(parts, axis=-1),
                                        jnp.int32)


_EB = 4000


def _filter_body(f_ref, rc_ref, wf1_ref, bf1_ref, wf2_ref, bf2_ref,
                 x_ref, win_ref, o_ref, h_ref):
    @pl.when(pl.program_id(0) == 0)
    def _():
        h_ref[...] = jnp.dot(x_ref[...], win_ref[...],
                             preferred_element_type=jnp.float32)

    t = jnp.dot(f_ref[...], wf1_ref[...], preferred_element_type=jnp.float32)
    t = _ssp(t + bf1_ref[...])
    w = jnp.dot(t, wf2_ref[...], preferred_element_type=jnp.float32) + bf2_ref[...]
    w4 = (w * rc_ref[...]).reshape(_EB // 80, 2, 40, D)
    lo = _pack_cols(w4[:, 0])
    hi = _pack_cols(w4[:, 1])
    o_ref[...] = jnp.concatenate([lo, hi], axis=-1).reshape(_EB // 2, D)


def _filter_call(f_ij, rcut_col, wf1, bf1, wf2, bf2, x, win):
    grid = E // _EB
    return pl.pallas_call(
        _filter_body,
        grid=(grid,),
        in_specs=[
            pl.BlockSpec((_EB, R), lambda i: (i, 0)),
            pl.BlockSpec((_EB, 1), lambda i: (i, 0)),
            pl.BlockSpec((R, D), lambda i: (0, 0)),
            pl.BlockSpec((1, D), lambda i: (0, 0)),
            pl.BlockSpec((D, D), lambda i: (0, 0)),
            pl.BlockSpec((1, D), lambda i: (0, 0)),
            pl.BlockSpec((N, D), lambda i: (0, 0)),
            pl.BlockSpec((D, D), lambda i: (0, 0)),
        ],
        out_specs=[
            pl.BlockSpec((_EB // 2, D), lambda i: (i, 0)),
            pl.BlockSpec((N, D), lambda i: (0, 0)),
        ],
        out_shape=[
            jax.ShapeDtypeStruct((E // 2, D), jnp.int32),
            jax.ShapeDtypeStruct((N, D), jnp.float32),
        ],
    )(f_ij, rcut_col, wf1, bf1, wf2, bf2, x, win)


_NC, _NS = 2, 16
_NW = _NC * _NS
_EPW = E // _NW
_K = 80
_NCHUNK = _EPW // _K
_NP = 10240
_RPT = _NP // _NS
_ZR = 128
_NPAIR = (_NCHUNK - 1) // 2


def _sc_body(h_hbm, wij_hbm, comb_hbm, out_hbm,
             comb_v, idxj0, idxj1, idxi0, idxi1, rows_v, wij_v, agg_sh,
             i0sem, i1sem, g0, g1, w0, w1, s0, s1):
    c = lax.axis_index("c")
    s = lax.axis_index("s")
    wid = s * _NC + c

    def _z1(i, carry):
        r = i // (D // 16)
        col = (i % (D // 16)) * 16
        rows_v[r, pl.ds(col, 16)] = jnp.zeros((16,), jnp.float32)
        return carry

    lax.fori_loop(0, _ZR * (D // 16), _z1, 0)
    r0 = s * _RPT
    zsrc = rows_v.at[pl.ds(0, _ZR)]

    def _zcopy(j, carry):
        pltpu.sync_copy(zsrc, agg_sh.at[pl.ds(r0 + j * _ZR, _ZR)])
        return carry

    lax.fori_loop(0, _RPT // _ZR, _zcopy, 0)
    plsc.subcore_barrier()

    ebase0 = wid * _EPW
    cb0 = wid * _NCHUNK

    def _extract(slot, dst, base):
        def _cp(i, cc):
            dst[pl.ds(i * 16, 16)] = comb_v[slot, 0, pl.ds(base + i * 16, 16)]
            return cc

        lax.fori_loop(0, _K // 16, _cp, 0, unroll=True)

    _KW = _K // 2

    def _fire(k, slot, idxj, gsem, wsem):
        pltpu.async_copy(h_hbm.at[idxj], rows_v.at[pl.ds(slot * _K, _K)], gsem)
        woff = pl.multiple_of((ebase0 + k * _K) // 2, 8)
        pltpu.async_copy(wij_hbm.at[pl.ds(woff, _KW)],
                         wij_v.at[pl.ds(slot * _KW, _KW)], wsem)

    def _wait_gw(slot, gsem, wsem):
        pltpu.make_async_copy(h_hbm.at[pl.ds(0, _K)],
                              rows_v.at[pl.ds(slot * _K, _K)], gsem).wait()
        pltpu.make_async_copy(wij_hbm.at[pl.ds(0, _KW)],
                              wij_v.at[pl.ds(slot * _KW, _KW)], wsem).wait()

    def _drain_s(sem):
        pltpu.make_async_copy(h_hbm.at[pl.ds(0, _K)],
                              rows_v.at[pl.ds(0, _K)], sem).wait()

    def _mul_chunk(slot):
        def _mul(i, cc):
            e = i // 4
            g = i % 4
            re = slot * _K + e
            ww = wij_v[slot * _KW + e % _KW,
                       pl.ds((e // _KW) * 64 + g * 16, 16)]
            wlo = lax.bitcast_convert_type(ww << 16, jnp.float32)
            whi = lax.bitcast_convert_type(ww & (-65536), jnp.float32)
            rows_v[re, pl.ds(32 * g, 16)] = (
                rows_v[re, pl.ds(32 * g, 16)] * wlo)
            rows_v[re, pl.ds(32 * g + 16, 16)] = (
                rows_v[re, pl.ds(32 * g + 16, 16)] * whi)
            return cc

        lax.fori_loop(0, _K * 4, _mul, 0, unroll=8)

    pltpu.sync_copy(comb_hbm.at[cb0], comb_v.at[0])
    pltpu.sync_copy(comb_hbm.at[cb0 + 1], comb_v.at[1])
    _extract(0, idxj0, 0)
    _extract(0, idxi0, _K)
    _extract(1, idxj1, 0)
    _extract(1, idxi1, _K)
    _fire(0, 0, idxj0, g0, w0)
    _fire(1, 1, idxj1, g1, w1)

    def _pair(blk, carry):
        k = 2 * blk
        pltpu.async_copy(comb_hbm.at[cb0 + k + 2], comb_v.at[0], i0sem)

        @pl.when(blk < _NPAIR - 1)
        def _():
            pltpu.async_copy(comb_hbm.at[cb0 + k + 3], comb_v.at[1], i1sem)

        _wait_gw(0, g0, w0)
        _mul_chunk(0)
        pltpu.async_copy(rows_v.at[pl.ds(0, _K)], agg_sh.at[idxi0], s0,
                         add=True)

        _wait_gw(1, g1, w1)
        _mul_chunk(1)
        pltpu.async_copy(rows_v.at[pl.ds(_K, _K)], agg_sh.at[idxi1], s1,
                         add=True)

        pltpu.make_async_copy(comb_hbm.at[cb0], comb_v.at[0], i0sem).wait()
        _drain_s(s0)
        _extract(0, idxj0, 0)
        _extract(0, idxi0, _K)
        _fire(k + 2, 0, idxj0, g0, w0)

        @pl.when(blk < _NPAIR - 1)
        def _():
            pltpu.make_async_copy(comb_hbm.at[cb0], comb_v.at[1], i1sem).wait()
            _drain_s(s1)
            _extract(1, idxj1, 0)
            _extract(1, idxi1, _K)
            _fire(k + 3, 1, idxj1, g1, w1)

        return carry

    lax.fori_loop(0, _NPAIR, _pair, 0)

    _wait_gw(0, g0, w0)
    _mul_chunk(0)
    pltpu.sync_copy(rows_v.at[pl.ds(0, _K)], agg_sh.at[idxi0], add=True)
    _drain_s(s1)
    plsc.subcore_barrier()

    def _dump(j, carry):
        rr = r0 + j * _ZR
        pltpu.sync_copy(agg_sh.at[pl.ds(rr, _ZR)], zsrc)
        pltpu.sync_copy(zsrc, out_hbm.at[pl.ds(c * _NP + rr, _ZR)])
        return carry

    lax.fori_loop(0, _RPT // _ZR, _dump, 0)


def _sc_conv(h, wij, comb):
    mesh = plsc.VectorSubcoreMesh(core_axis_name="c", subcore_axis_name="s")
    fn = pl.kernel(
        _sc_body,
        out_type=jax.ShapeDtypeStruct((_NC * _NP, D), jnp.float32),
        mesh=mesh,
        scratch_types=[
            pltpu.VMEM((2, 1, 2 * _K), jnp.int32),
            pltpu.VMEM((_K,), jnp.int32),
            pltpu.VMEM((_K,), jnp.int32),
            pltpu.VMEM((_K,), jnp.int32),
            pltpu.VMEM((_K,), jnp.int32),
            pltpu.VMEM((2 * _K, D), jnp.float32),
            pltpu.VMEM((_K, D), jnp.int32),
            pltpu.VMEM_SHARED((_NP, D), jnp.float32),
            pltpu.SemaphoreType.DMA,
            pltpu.SemaphoreType.DMA,
            pltpu.SemaphoreType.DMA,
            pltpu.SemaphoreType.DMA,
            pltpu.SemaphoreType.DMA,
            pltpu.SemaphoreType.DMA,
            pltpu.SemaphoreType.DMA,
            pltpu.SemaphoreType.DMA,
        ],
    )
    return fn(h, wij, comb)


def _out_body(p_ref, wo1_ref, bo1_ref, wo2_ref, bo2_ref, o_ref):
    agg = p_ref[:N, :] + p_ref[_NP:_NP + N, :]
    t = _ssp(jnp.dot(agg, wo1_ref[...], preferred_element_type=jnp.float32)
             + bo1_ref[...])
    o_ref[...] = (jnp.dot(t, wo2_ref[...], preferred_element_type=jnp.float32)
                  + bo2_ref[...])


def _out_call(partials, wo1, bo1, wo2, bo2):
    return pl.pallas_call(
        _out_body,
        out_shape=jax.ShapeDtypeStruct((N, D), jnp.float32),
    )(partials, wo1, bo1, wo2, bo2)


def kernel(x, f_ij, idx_i, idx_j, rcut_ij,
           W_in2f, W_f1, b_f1, W_f2, b_f2, W_o1, b_o1, W_o2, b_o2):
    idx_i32 = idx_i.astype(jnp.int32)
    idx_j32 = idx_j.astype(jnp.int32)
    comb = jnp.concatenate(
        [idx_j32.reshape(E // _K, 1, _K), idx_i32.reshape(E // _K, 1, _K)],
        axis=2)
    wij, h = _filter_call(f_ij, rcut_ij.reshape(E, 1),
                          W_f1, b_f1.reshape(1, D), W_f2, b_f2.reshape(1, D),
                          x, W_in2f)
    partials = _sc_conv(h, wij, comb)
    return _out_call(partials, W_o1, b_o1.reshape(1, D),
                     W_o2, b_o2.reshape(1, D))


# speedup vs baseline: 2.0843x; 1.0065x over previous
import functools

import jax
import jax.numpy as jnp
from jax import lax
from jax.experimental import pallas as pl
from jax.experimental.pallas import tpu as pltpu
from jax.experimental.pallas import tpu_sc as plsc

N, E, D, R = 10000, 320000, 128, 16

_LOG2 = 0.6931471805599453


def _ssp(x):
    return jnp.maximum(x, 0.0) + jnp.log(1.0 + jnp.exp(-jnp.abs(x))) - _LOG2


def _pack_cols(w):
    def rne(x):
        r = jax.lax.bitcast_convert_type(x, jnp.uint32)
        return (r + jnp.uint32(0x7FFF) + ((r >> 16) & jnp.uint32(1))) >> 16

    parts = []
    for g in range(4):
        lo = rne(w[..., 32 * g:32 * g + 16])
        hi = rne(w[..., 32 * g + 16:32 * g + 32])
        parts.append((hi << 16) | lo)
    return jax.lax.bitcast_convert_type(jnp.concatenate(parts, axis=-1),
                                        jnp.int32)


_EB = 4000


def _filter_body(f_ref, rc_ref, wf1_ref, bf1_ref, wf2_ref, bf2_ref,
                 x_ref, win_ref, o_ref, h_ref):
    @pl.when(pl.program_id(0) == 0)
    def _():
        h_ref[...] = jnp.dot(x_ref[...], win_ref[...],
                             preferred_element_type=jnp.float32)

    t = jnp.dot(f_ref[...], wf1_ref[...], preferred_element_type=jnp.float32)
    t = _ssp(t + bf1_ref[...])
    w = jnp.dot(t, wf2_ref[...], preferred_element_type=jnp.float32) + bf2_ref[...]
    w4 = (w * rc_ref[...]).reshape(_EB // 80, 2, 40, D)
    lo = _pack_cols(w4[:, 0])
    hi = _pack_cols(w4[:, 1])
    o_ref[...] = jnp.concatenate([lo, hi], axis=-1).reshape(_EB // 2, D)


def _filter_call(f_ij, rcut_col, wf1, bf1, wf2, bf2, x, win):
    grid = E // _EB
    return pl.pallas_call(
        _filter_body,
        grid=(grid,),
        in_specs=[
            pl.BlockSpec((_EB, R), lambda i: (i, 0)),
            pl.BlockSpec((_EB, 1), lambda i: (i, 0)),
            pl.BlockSpec((R, D), lambda i: (0, 0)),
            pl.BlockSpec((1, D), lambda i: (0, 0)),
            pl.BlockSpec((D, D), lambda i: (0, 0)),
            pl.BlockSpec((1, D), lambda i: (0, 0)),
            pl.BlockSpec((N, D), lambda i: (0, 0)),
            pl.BlockSpec((D, D), lambda i: (0, 0)),
        ],
        out_specs=[
            pl.BlockSpec((_EB // 2, D), lambda i: (i, 0)),
            pl.BlockSpec((N, D), lambda i: (0, 0)),
        ],
        out_shape=[
            jax.ShapeDtypeStruct((E // 2, D), jnp.int32),
            jax.ShapeDtypeStruct((N, D), jnp.float32),
        ],
    )(f_ij, rcut_col, wf1, bf1, wf2, bf2, x, win)


_NC, _NS = 2, 16
_NW = _NC * _NS
_EPW = E // _NW
_K = 80
_NCHUNK = _EPW // _K
_NP = 10240
_RPT = _NP // _NS
_ZR = 128
_NPAIR = (_NCHUNK - 1) // 2


def _sc_body(h_hbm, wij_hbm, comb_hbm, out_hbm,
             comb_v, idxj0, idxj1, idxi0, idxi1, rows_v, wij_v, agg_sh,
             i0sem, i1sem, g0, g1, w0, w1, s0, s1):
    c = lax.axis_index("c")
    s = lax.axis_index("s")
    wid = s * _NC + c

    def _z1(i, carry):
        r = i // (D // 16)
        col = (i % (D // 16)) * 16
        rows_v[r, pl.ds(col, 16)] = jnp.zeros((16,), jnp.float32)
        return carry

    lax.fori_loop(0, _ZR * (D // 16), _z1, 0)
    r0 = s * _RPT
    zsrc = rows_v.at[pl.ds(0, _ZR)]

    def _zcopy(j, carry):
        pltpu.sync_copy(zsrc, agg_sh.at[pl.ds(r0 + j * _ZR, _ZR)])
        return carry

    lax.fori_loop(0, _RPT // _ZR, _zcopy, 0)
    plsc.subcore_barrier()

    ebase0 = wid * _EPW
    cb0 = wid * _NCHUNK

    def _extract(slot, dst, row):
        def _cp(i, cc):
            dst[pl.ds(i * 16, 16)] = comb_v[slot, row, pl.ds(i * 16, 16)]
            return cc

        lax.fori_loop(0, _K // 16, _cp, 0, unroll=True)

    _KW = _K // 2

    def _fire(k, slot, idxj, gsem, wsem):
        pltpu.async_copy(h_hbm.at[idxj], rows_v.at[pl.ds(slot * _K, _K)], gsem)
        woff = pl.multiple_of((ebase0 + k * _K) // 2, 8)
        pltpu.async_copy(wij_hbm.at[pl.ds(woff, _KW)],
                         wij_v.at[pl.ds(slot * _KW, _KW)], wsem)

    def _wait_gw(slot, gsem, wsem):
        pltpu.make_async_copy(h_hbm.at[pl.ds(0, _K)],
                              rows_v.at[pl.ds(slot * _K, _K)], gsem).wait()
        pltpu.make_async_copy(wij_hbm.at[pl.ds(0, _KW)],
                              wij_v.at[pl.ds(slot * _KW, _KW)], wsem).wait()

    def _drain_s(sem):
        pltpu.make_async_copy(h_hbm.at[pl.ds(0, _K)],
                              rows_v.at[pl.ds(0, _K)], sem).wait()

    def _mul_chunk(slot):
        def _mul(i, cc):
            e = i // 4
            g = i % 4
            re = slot * _K + e
            ww = wij_v[slot * _KW + e % _KW,
                       pl.ds((e // _KW) * 64 + g * 16, 16)]
            wlo = lax.bitcast_convert_type(ww << 16, jnp.float32)
            whi = lax.bitcast_convert_type(ww & (-65536), jnp.float32)
            rows_v[re, pl.ds(32 * g, 16)] = (
                rows_v[re, pl.ds(32 * g, 16)] * wlo)
            rows_v[re, pl.ds(32 * g + 16, 16)] = (
                rows_v[re, pl.ds(32 * g + 16, 16)] * whi)
            return cc

        lax.fori_loop(0, _K * 4, _mul, 0, unroll=8)

    pltpu.sync_copy(comb_hbm.at[cb0], comb_v.at[0])
    pltpu.sync_copy(comb_hbm.at[cb0 + 1], comb_v.at[1])
    _extract(0, idxj0, 0)
    _extract(0, idxi0, 1)
    _extract(1, idxj1, 0)
    _extract(1, idxi1, 1)
    _fire(0, 0, idxj0, g0, w0)
    _fire(1, 1, idxj1, g1, w1)

    def _pair(blk, carry):
        k = 2 * blk
        pltpu.async_copy(comb_hbm.at[cb0 + k + 2], comb_v.at[0], i0sem)

        @pl.when(blk < _NPAIR - 1)
        def _():
            pltpu.async_copy(comb_hbm.at[cb0 + k + 3], comb_v.at[1], i1sem)

        _wait_gw(0, g0, w0)
        _mul_chunk(0)
        pltpu.async_copy(rows_v.at[pl.ds(0, _K)], agg_sh.at[idxi0], s0,
                         add=True)

        _wait_gw(1, g1, w1)
        _mul_chunk(1)
        pltpu.async_copy(rows_v.at[pl.ds(_K, _K)], agg_sh.at[idxi1], s1,
                         add=True)

        pltpu.make_async_copy(comb_hbm.at[cb0], comb_v.at[0], i0sem).wait()
        _drain_s(s0)
        _extract(0, idxj0, 0)
        _extract(0, idxi0, 1)
        _fire(k + 2, 0, idxj0, g0, w0)

        @pl.when(blk < _NPAIR - 1)
        def _():
            pltpu.make_async_copy(comb_hbm.at[cb0], comb_v.at[1], i1sem).wait()
            _drain_s(s1)
            _extract(1, idxj1, 0)
            _extract(1, idxi1, 1)
            _fire(k + 3, 1, idxj1, g1, w1)

        return carry

    lax.fori_loop(0, _NPAIR, _pair, 0)

    _wait_gw(0, g0, w0)
    _mul_chunk(0)
    pltpu.sync_copy(rows_v.at[pl.ds(0, _K)], agg_sh.at[idxi0], add=True)
    _drain_s(s1)
    plsc.subcore_barrier()

    def _dump(j, carry):
        rr = r0 + j * _ZR
        pltpu.sync_copy(agg_sh.at[pl.ds(rr, _ZR)], zsrc)
        pltpu.sync_copy(zsrc, out_hbm.at[pl.ds(c * _NP + rr, _ZR)])
        return carry

    lax.fori_loop(0, _RPT // _ZR, _dump, 0)


def _sc_conv(h, wij, comb):
    mesh = plsc.VectorSubcoreMesh(core_axis_name="c", subcore_axis_name="s")
    fn = pl.kernel(
        _sc_body,
        out_type=jax.ShapeDtypeStruct((_NC * _NP, D), jnp.float32),
        mesh=mesh,
        scratch_types=[
            pltpu.VMEM((2, 2, 128), jnp.int32),
            pltpu.VMEM((_K,), jnp.int32),
            pltpu.VMEM((_K,), jnp.int32),
            pltpu.VMEM((_K,), jnp.int32),
            pltpu.VMEM((_K,), jnp.int32),
            pltpu.VMEM((2 * _K, D), jnp.float32),
            pltpu.VMEM((_K, D), jnp.int32),
            pltpu.VMEM_SHARED((_NP, D), jnp.float32),
            pltpu.SemaphoreType.DMA,
            pltpu.SemaphoreType.DMA,
            pltpu.SemaphoreType.DMA,
            pltpu.SemaphoreType.DMA,
            pltpu.SemaphoreType.DMA,
            pltpu.SemaphoreType.DMA,
            pltpu.SemaphoreType.DMA,
            pltpu.SemaphoreType.DMA,
        ],
    )
    return fn(h, wij, comb)


def _out_body(p_ref, wo1_ref, bo1_ref, wo2_ref, bo2_ref, o_ref):
    agg = p_ref[:N, :] + p_ref[_NP:_NP + N, :]
    t = _ssp(jnp.dot(agg, wo1_ref[...], preferred_element_type=jnp.float32)
             + bo1_ref[...])
    o_ref[...] = (jnp.dot(t, wo2_ref[...], preferred_element_type=jnp.float32)
                  + bo2_ref[...])


def _out_call(partials, wo1, bo1, wo2, bo2):
    return pl.pallas_call(
        _out_body,
        out_shape=jax.ShapeDtypeStruct((N, D), jnp.float32),
    )(partials, wo1, bo1, wo2, bo2)


def kernel(x, f_ij, idx_i, idx_j, rcut_ij,
           W_in2f, W_f1, b_f1, W_f2, b_f2, W_o1, b_o1, W_o2, b_o2):
    idx_i32 = idx_i.astype(jnp.int32)
    idx_j32 = idx_j.astype(jnp.int32)
    comb = jnp.stack(
        [jnp.pad(idx_j32.reshape(E // _K, _K), ((0, 0), (0, 128 - _K))),
         jnp.pad(idx_i32.reshape(E // _K, _K), ((0, 0), (0, 128 - _K)))],
        axis=1)
    wij, h = _filter_call(f_ij, rcut_ij.reshape(E, 1),
                          W_f1, b_f1.reshape(1, D), W_f2, b_f2.reshape(1, D),
                          x, W_in2f)
    partials = _sc_conv(h, wij, comb)
    return _out_call(partials, W_o1, b_o1.reshape(1, D),
                     W_o2, b_o2.reshape(1, D))


# speedup vs baseline: 2.4145x; 1.1584x over previous
import functools

import jax
import jax.numpy as jnp
from jax import lax
from jax.experimental import pallas as pl
from jax.experimental.pallas import tpu as pltpu
from jax.experimental.pallas import tpu_sc as plsc

N, E, D, R = 10000, 320000, 128, 16

_LOG2 = 0.6931471805599453


def _ssp(x):
    return jnp.maximum(x, 0.0) + jnp.log(1.0 + jnp.exp(-jnp.abs(x))) - _LOG2


def _pack_cols(w):
    def rne(x):
        r = jax.lax.bitcast_convert_type(x, jnp.uint32)
        return (r + jnp.uint32(0x7FFF) + ((r >> 16) & jnp.uint32(1))) >> 16

    parts = []
    for g in range(4):
        lo = rne(w[..., 32 * g:32 * g + 16])
        hi = rne(w[..., 32 * g + 16:32 * g + 32])
        parts.append((hi << 16) | lo)
    return jax.lax.bitcast_convert_type(jnp.concatenate(parts, axis=-1),
                                        jnp.int32)


_EB = 3200


def _filter_body(f_ref, rc_ref, wf1_ref, bf1_ref, wf2_ref, bf2_ref,
                 x_ref, win_ref, o_ref, h_ref):
    @pl.when(pl.program_id(0) == 0)
    def _():
        h_ref[...] = jnp.dot(x_ref[...], win_ref[...],
                             preferred_element_type=jnp.float32)

    t = jnp.dot(f_ref[...], wf1_ref[...], preferred_element_type=jnp.float32)
    t = _ssp(t + bf1_ref[...])
    w = jnp.dot(t, wf2_ref[...], preferred_element_type=jnp.float32) + bf2_ref[...]
    w3 = w.reshape(_EB // 128, 128, D) * rc_ref[0][:, :, None]
    w4 = w3.reshape(_EB // 80, 2, 40, D)
    lo = _pack_cols(w4[:, 0])
    hi = _pack_cols(w4[:, 1])
    o_ref[...] = jnp.concatenate([lo, hi], axis=-1).reshape(_EB // 2, D)


def _filter_call(f_ij, rcut_rows, wf1, bf1, wf2, bf2, x, win):
    grid = E // _EB
    return pl.pallas_call(
        _filter_body,
        grid=(grid,),
        in_specs=[
            pl.BlockSpec((_EB, R), lambda i: (i, 0)),
            pl.BlockSpec((1, _EB // 128, 128), lambda i: (i, 0, 0)),
            pl.BlockSpec((R, D), lambda i: (0, 0)),
            pl.BlockSpec((1, D), lambda i: (0, 0)),
            pl.BlockSpec((D, D), lambda i: (0, 0)),
            pl.BlockSpec((1, D), lambda i: (0, 0)),
            pl.BlockSpec((N, D), lambda i: (0, 0)),
            pl.BlockSpec((D, D), lambda i: (0, 0)),
        ],
        out_specs=[
            pl.BlockSpec((_EB // 2, D), lambda i: (i, 0)),
            pl.BlockSpec((N, D), lambda i: (0, 0)),
        ],
        out_shape=[
            jax.ShapeDtypeStruct((E // 2, D), jnp.int32),
            jax.ShapeDtypeStruct((N, D), jnp.float32),
        ],
    )(f_ij, rcut_rows, wf1, bf1, wf2, bf2, x, win)


_NC, _NS = 2, 16
_NW = _NC * _NS
_EPW = E // _NW
_K = 80
_NCHUNK = _EPW // _K
_NP = 10240
_RPT = _NP // _NS
_ZR = 128
_NPAIR = (_NCHUNK - 1) // 2


def _sc_body(h_hbm, wij_hbm, comb_hbm, out_hbm,
             comb_v, idxj0, idxj1, idxi0, idxi1, rows_v, wij_v, agg_sh,
             i0sem, i1sem, g0, g1, w0, w1, s0, s1):
    c = lax.axis_index("c")
    s = lax.axis_index("s")
    wid = s * _NC + c

    def _z1(i, carry):
        r = i // (D // 16)
        col = (i % (D // 16)) * 16
        rows_v[r, pl.ds(col, 16)] = jnp.zeros((16,), jnp.float32)
        return carry

    lax.fori_loop(0, _ZR * (D // 16), _z1, 0)
    r0 = s * _RPT
    zsrc = rows_v.at[pl.ds(0, _ZR)]

    def _zcopy(j, carry):
        pltpu.sync_copy(zsrc, agg_sh.at[pl.ds(r0 + j * _ZR, _ZR)])
        return carry

    lax.fori_loop(0, _RPT // _ZR, _zcopy, 0)
    plsc.subcore_barrier()

    ebase0 = wid * _EPW
    cb0 = wid * _NCHUNK

    def _extract(slot, dst, row):
        def _cp(i, cc):
            dst[pl.ds(i * 16, 16)] = comb_v[slot, row, pl.ds(i * 16, 16)]
            return cc

        lax.fori_loop(0, _K // 16, _cp, 0, unroll=True)

    _KW = _K // 2

    def _fire(k, slot, idxj, gsem, wsem):
        pltpu.async_copy(h_hbm.at[idxj], rows_v.at[pl.ds(slot * _K, _K)], gsem)
        woff = pl.multiple_of((ebase0 + k * _K) // 2, 8)
        pltpu.async_copy(wij_hbm.at[pl.ds(woff, _KW)],
                         wij_v.at[pl.ds(slot * _KW, _KW)], wsem)

    def _wait_gw(slot, gsem, wsem):
        pltpu.make_async_copy(h_hbm.at[pl.ds(0, _K)],
                              rows_v.at[pl.ds(slot * _K, _K)], gsem).wait()
        pltpu.make_async_copy(wij_hbm.at[pl.ds(0, _KW)],
                              wij_v.at[pl.ds(slot * _KW, _KW)], wsem).wait()

    def _drain_s(sem):
        pltpu.make_async_copy(h_hbm.at[pl.ds(0, _K)],
                              rows_v.at[pl.ds(0, _K)], sem).wait()

    def _mul_chunk(slot):
        def _mul(i, cc):
            e = i // 4
            g = i % 4
            re = slot * _K + e
            ww = wij_v[slot * _KW + e % _KW,
                       pl.ds((e // _KW) * 64 + g * 16, 16)]
            wlo = lax.bitcast_convert_type(ww << 16, jnp.float32)
            whi = lax.bitcast_convert_type(ww & (-65536), jnp.float32)
            rows_v[re, pl.ds(32 * g, 16)] = (
                rows_v[re, pl.ds(32 * g, 16)] * wlo)
            rows_v[re, pl.ds(32 * g + 16, 16)] = (
                rows_v[re, pl.ds(32 * g + 16, 16)] * whi)
            return cc

        lax.fori_loop(0, _K * 4, _mul, 0, unroll=8)

    pltpu.sync_copy(comb_hbm.at[cb0], comb_v.at[0])
    pltpu.sync_copy(comb_hbm.at[cb0 + 1], comb_v.at[1])
    _extract(0, idxj0, 0)
    _extract(0, idxi0, 1)
    _extract(1, idxj1, 0)
    _extract(1, idxi1, 1)
    _fire(0, 0, idxj0, g0, w0)
    _fire(1, 1, idxj1, g1, w1)

    def _pair(blk, carry):
        k = 2 * blk
        pltpu.async_copy(comb_hbm.at[cb0 + k + 2], comb_v.at[0], i0sem)

        @pl.when(blk < _NPAIR - 1)
        def _():
            pltpu.async_copy(comb_hbm.at[cb0 + k + 3], comb_v.at[1], i1sem)

        _wait_gw(0, g0, w0)
        _mul_chunk(0)
        pltpu.async_copy(rows_v.at[pl.ds(0, _K)], agg_sh.at[idxi0], s0,
                         add=True)

        _wait_gw(1, g1, w1)
        _mul_chunk(1)
        pltpu.async_copy(rows_v.at[pl.ds(_K, _K)], agg_sh.at[idxi1], s1,
                         add=True)

        pltpu.make_async_copy(comb_hbm.at[cb0], comb_v.at[0], i0sem).wait()
        _drain_s(s0)
        _extract(0, idxj0, 0)
        _extract(0, idxi0, 1)
        _fire(k + 2, 0, idxj0, g0, w0)

        @pl.when(blk < _NPAIR - 1)
        def _():
            pltpu.make_async_copy(comb_hbm.at[cb0], comb_v.at[1], i1sem).wait()
            _drain_s(s1)
            _extract(1, idxj1, 0)
            _extract(1, idxi1, 1)
            _fire(k + 3, 1, idxj1, g1, w1)

        return carry

    lax.fori_loop(0, _NPAIR, _pair, 0)

    _wait_gw(0, g0, w0)
    _mul_chunk(0)
    pltpu.sync_copy(rows_v.at[pl.ds(0, _K)], agg_sh.at[idxi0], add=True)
    _drain_s(s1)
    plsc.subcore_barrier()

    def _dump(j, carry):
        rr = r0 + j * _ZR
        pltpu.sync_copy(agg_sh.at[pl.ds(rr, _ZR)], zsrc)
        pltpu.sync_copy(zsrc, out_hbm.at[pl.ds(c * _NP + rr, _ZR)])
        return carry

    lax.fori_loop(0, _RPT // _ZR, _dump, 0)


def _sc_conv(h, wij, comb):
    mesh = plsc.VectorSubcoreMesh(core_axis_name="c", subcore_axis_name="s")
    fn = pl.kernel(
        _sc_body,
        out_type=jax.ShapeDtypeStruct((_NC * _NP, D), jnp.float32),
        mesh=mesh,
        scratch_types=[
            pltpu.VMEM((2, 2, 128), jnp.int32),
            pltpu.VMEM((_K,), jnp.int32),
            pltpu.VMEM((_K,), jnp.int32),
            pltpu.VMEM((_K,), jnp.int32),
            pltpu.VMEM((_K,), jnp.int32),
            pltpu.VMEM((2 * _K, D), jnp.float32),
            pltpu.VMEM((_K, D), jnp.int32),
            pltpu.VMEM_SHARED((_NP, D), jnp.float32),
            pltpu.SemaphoreType.DMA,
            pltpu.SemaphoreType.DMA,
            pltpu.SemaphoreType.DMA,
            pltpu.SemaphoreType.DMA,
            pltpu.SemaphoreType.DMA,
            pltpu.SemaphoreType.DMA,
            pltpu.SemaphoreType.DMA,
            pltpu.SemaphoreType.DMA,
        ],
    )
    return fn(h, wij, comb)


def _out_body(p_ref, wo1_ref, bo1_ref, wo2_ref, bo2_ref, o_ref):
    agg = p_ref[:N, :] + p_ref[_NP:_NP + N, :]
    t = _ssp(jnp.dot(agg, wo1_ref[...], preferred_element_type=jnp.float32)
             + bo1_ref[...])
    o_ref[...] = (jnp.dot(t, wo2_ref[...], preferred_element_type=jnp.float32)
                  + bo2_ref[...])


def _out_call(partials, wo1, bo1, wo2, bo2):
    return pl.pallas_call(
        _out_body,
        out_shape=jax.ShapeDtypeStruct((N, D), jnp.float32),
    )(partials, wo1, bo1, wo2, bo2)


def kernel(x, f_ij, idx_i, idx_j, rcut_ij,
           W_in2f, W_f1, b_f1, W_f2, b_f2, W_o1, b_o1, W_o2, b_o2):
    idx_i32 = idx_i.astype(jnp.int32)
    idx_j32 = idx_j.astype(jnp.int32)
    comb = jnp.stack(
        [jnp.pad(idx_j32.reshape(E // _K, _K), ((0, 0), (0, 128 - _K))),
         jnp.pad(idx_i32.reshape(E // _K, _K), ((0, 0), (0, 128 - _K)))],
        axis=1)
    wij, h = _filter_call(f_ij, rcut_ij.reshape(E // _EB, _EB // 128, 128),
                          W_f1, b_f1.reshape(1, D), W_f2, b_f2.reshape(1, D),
                          x, W_in2f)
    partials = _sc_conv(h, wij, comb)
    return _out_call(partials, W_o1, b_o1.reshape(1, D),
                     W_o2, b_o2.reshape(1, D))
